# Initial kernel scaffold; baseline (speedup 1.0000x reference)
import jax
import jax.numpy as jnp
from jax.experimental import pallas as pl


def kernel(x, tables):
    raise NotImplementedError("write your pallas kernel here")


# speedup vs baseline: 1.1831x; 1.1831x over previous
import functools

import jax
import jax.numpy as jnp
from jax import lax
from jax.experimental import pallas as pl
from jax.experimental.pallas import tpu as pltpu
from jax.experimental.pallas import tpu_sc as plsc

_NC = 2
_NS = 16
_NW = _NC * _NS

_IDX_PER_GATHER = 128


def _sc_embed_sum(B, F, H, R):
    RW = B // _NW
    CH = RW // R
    IPC = R * F
    NG = IPC // _IDX_PER_GATHER
    assert RW * _NW == B and CH * R == RW and NG * _IDX_PER_GATHER == IPC
    assert H % 16 == 0

    mesh = plsc.VectorSubcoreMesh(core_axis_name="c", subcore_axis_name="s")

    @functools.partial(
        pl.kernel,
        mesh=mesh,
        out_type=jax.ShapeDtypeStruct((B, H), jnp.float32),
        scratch_types=[
            pltpu.VMEM((IPC,), jnp.int32),
            pltpu.VMEM((IPC, H), jnp.float32),
            pltpu.VMEM((R, H), jnp.float32),
            pltpu.SemaphoreType.DMA,
        ],
        compiler_params=pltpu.CompilerParams(use_tc_tiling_on_sc=False),
    )
    def k(idx_hbm, tab_hbm, out_hbm, idx_v, rows_v, acc_v, gsem):
        wid = lax.axis_index("s") * _NC + lax.axis_index("c")

        def chunk_body(c, carry):
            out_row0 = wid * RW + c * R
            idx_off = wid * (CH * IPC) + c * IPC
            pltpu.sync_copy(idx_hbm.at[pl.ds(idx_off, IPC)], idx_v)
            copies = []
            for j in range(NG):
                copies.append(
                    pltpu.async_copy(
                        tab_hbm.at[idx_v.at[pl.ds(j * _IDX_PER_GATHER, _IDX_PER_GATHER)]],
                        rows_v.at[pl.ds(j * _IDX_PER_GATHER, _IDX_PER_GATHER)],
                        gsem,
                    )
                )
            for cp in copies:
                cp.wait()

            def row_body(b, carry2):
                r0 = b * F
                accs = [rows_v[r0, pl.ds(16 * h, 16)] for h in range(H // 16)]
                for kf in range(1, F):
                    for h in range(H // 16):
                        accs[h] = accs[h] + rows_v[r0 + kf, pl.ds(16 * h, 16)]
                for h in range(H // 16):
                    acc_v[b, pl.ds(16 * h, 16)] = accs[h]
                return carry2

            lax.fori_loop(0, R, row_body, 0, unroll=False)
            pltpu.sync_copy(acc_v, out_hbm.at[pl.ds(out_row0, R)])
            return carry

        lax.fori_loop(0, CH, chunk_body, 0, unroll=False)

    return k


def kernel(x, tables):
    if x.ndim == 1:
        x = x[:, None]
    B, F = x.shape
    Ftab, V, H = tables.shape
    assert F == Ftab
    flat_idx = x.astype(jnp.int32) + (jnp.arange(F, dtype=jnp.int32) * V)[None, :]
    flat_idx = flat_idx.reshape(B * F)
    flat_tab = tables.reshape(F * V, H)
    k = _sc_embed_sum(B, F, H, R=64)
    return k(flat_idx, flat_tab)
